# baseline (speedup 1.0000x reference)
import math

import jax
import jax.numpy as jnp
from jax import lax
from jax.experimental import pallas as pl
from jax.experimental.pallas import tpu as pltpu
from jax.experimental.pallas import tpu_sc as plsc

NC = 2
NS = 16
NW = NC * NS
CH = 64
NB = 4


def _sc_segsum(n_chunks, np_rows, d, with_deg, table, gidx, sidx):
    rows_per_tile = np_rows // NS
    zch = 64
    ngroups = n_chunks // NB

    out_type = [jax.ShapeDtypeStruct((NC, np_rows, d), jnp.float32)]
    if with_deg:
        out_type.append(jax.ShapeDtypeStruct((NC, np_rows), jnp.float32))

    scratch = [
        pltpu.VMEM((zch, d), jnp.float32),
        [pltpu.VMEM((CH,), jnp.int32) for _ in range(NB)],
        [pltpu.VMEM((CH,), jnp.int32) for _ in range(NB)],
        [pltpu.VMEM((CH, d), jnp.float32) for _ in range(NB)],
        pltpu.VMEM_SHARED((np_rows, d), jnp.float32),
        [pltpu.SemaphoreType.DMA for _ in range(NB)],
        [pltpu.SemaphoreType.DMA for _ in range(NB)],
        [pltpu.SemaphoreType.DMA for _ in range(NB)],
        [pltpu.SemaphoreType.DMA for _ in range(NB)],
    ]
    if with_deg:
        scratch += [
            pltpu.VMEM((CH,), jnp.float32),
            pltpu.VMEM((rows_per_tile,), jnp.float32),
            pltpu.VMEM_SHARED((np_rows,), jnp.float32),
            [pltpu.SemaphoreType.DMA for _ in range(NB)],
        ]

    def body(table_hbm, gidx_hbm, sidx_hbm, *refs):
        if with_deg:
            (out_hbm, outd_hbm, zrows, gib, sib, rows, acc,
             gsem, ssem, gisem, sisem, ones_v, zdeg_v, dacc, dsem) = refs
        else:
            (out_hbm, zrows, gib, sib, rows, acc,
             gsem, ssem, gisem, sisem) = refs

        cid = lax.axis_index("c")
        sid = lax.axis_index("s")
        wid = sid * NC + cid
        base = sid * rows_per_tile

        def zrow(i, _):
            for j in range(d // 16):
                zrows[i, pl.ds(j * 16, 16)] = jnp.zeros((16,), jnp.float32)
            return 0
        lax.fori_loop(0, zch, zrow, 0)
        for k in range(rows_per_tile // zch):
            pltpu.sync_copy(zrows, acc.at[pl.ds(base + k * zch, zch)])
        if with_deg:
            def zdeg(i, _):
                zdeg_v[pl.ds(i * 16, 16)] = jnp.zeros((16,), jnp.float32)
                return 0
            lax.fori_loop(0, rows_per_tile // 16, zdeg, 0)
            for j in range(CH // 16):
                ones_v[pl.ds(j * 16, 16)] = jnp.ones((16,), jnp.float32)
            pltpu.sync_copy(zdeg_v, dacc.at[pl.ds(base, rows_per_tile)])

        def load_gidx(j, b):
            return pltpu.async_copy(gidx_hbm.at[wid, j], gib[b], gisem[b])

        def load_sidx(j, b):
            return pltpu.async_copy(sidx_hbm.at[wid, j], sib[b], sisem[b])

        def start_gather(b):
            return pltpu.async_copy(table_hbm.at[gib[b]], rows[b], gsem[b])

        def wait_gather(b):
            pltpu.make_async_copy(table_hbm.at[gib[b]], rows[b], gsem[b]).wait()

        def start_scatter(b):
            return pltpu.async_copy(rows[b], acc.at[sib[b]], ssem[b], add=True)

        def wait_scatter(b):
            pltpu.make_async_copy(rows[b], acc.at[sib[b]], ssem[b]).wait()

        def start_deg(b):
            return pltpu.async_copy(ones_v, dacc.at[sib[b]], dsem[b], add=True)

        def wait_deg(b):
            pltpu.make_async_copy(ones_v, dacc.at[sib[b]], dsem[b]).wait()

        def wait_gidx(j, b):
            pltpu.make_async_copy(gidx_hbm.at[wid, j], gib[b], gisem[b]).wait()

        def wait_sidx(j, b):
            pltpu.make_async_copy(sidx_hbm.at[wid, j], sib[b], sisem[b]).wait()

        plsc.subcore_barrier()

        for b in range(NB):
            load_gidx(b, b)
            load_sidx(b, b)
        for b in range(NB):
            wait_gidx(b, b)
            start_gather(b)

        def step(jo, _):
            j0 = NB * jo
            for b in range(NB):
                wait_gather(b)
                load_gidx(j0 + b + NB, b)
                wait_sidx(j0 + b, b)
                start_scatter(b)
                if with_deg:
                    start_deg(b)
            for b in range(NB):
                wait_scatter(b)
                if with_deg:
                    wait_deg(b)
                load_sidx(j0 + b + NB, b)
                wait_gidx(j0 + b + NB, b)
                start_gather(b)
            return 0
        lax.fori_loop(0, ngroups - 1, step, 0)

        jl = n_chunks - NB
        for b in range(NB):
            wait_gather(b)
            wait_sidx(jl + b, b)
            start_scatter(b)
            if with_deg:
                start_deg(b)
        for b in range(NB):
            wait_scatter(b)
            if with_deg:
                wait_deg(b)

        plsc.subcore_barrier()

        for k in range(rows_per_tile // zch):
            pltpu.sync_copy(acc.at[pl.ds(base + k * zch, zch)],
                            out_hbm.at[cid, pl.ds(base + k * zch, zch)])
        if with_deg:
            pltpu.sync_copy(dacc.at[pl.ds(base, rows_per_tile)],
                            outd_hbm.at[cid, pl.ds(base, rows_per_tile)])

    mesh = plsc.VectorSubcoreMesh(core_axis_name="c", subcore_axis_name="s",
                                  num_cores=NC, num_subcores=NS)
    run = pl.kernel(body, out_type=out_type, mesh=mesh, scratch_types=scratch)
    res = run(table, gidx, sidx)
    return res if with_deg else res[0]


def _tc_layer(n, d, ph, pr, degm, h, W, b8, L):
    bn = 1000
    grid = n // bn

    def body(ph_ref, pr_ref, degm_ref, h_ref, W_ref, b_ref, L_ref, o_ref):
        agg = (ph_ref[0] + ph_ref[1] + pr_ref[0] + pr_ref[1]) / degm_ref[...]
        o = jnp.dot(agg, W_ref[...], preferred_element_type=jnp.float32)
        o = o + jnp.dot(h_ref[...], L_ref[...], preferred_element_type=jnp.float32)
        o = o + b_ref[0:1, :]
        o_ref[...] = jnp.maximum(o, 0.0)

    return pl.pallas_call(
        body,
        grid=(grid,),
        in_specs=[
            pl.BlockSpec((NC, bn, d), lambda i: (0, i, 0)),
            pl.BlockSpec((NC, bn, d), lambda i: (0, i, 0)),
            pl.BlockSpec((bn, d), lambda i: (i, 0)),
            pl.BlockSpec((bn, d), lambda i: (i, 0)),
            pl.BlockSpec((d, d), lambda i: (0, 0)),
            pl.BlockSpec((8, d), lambda i: (0, 0)),
            pl.BlockSpec((d, d), lambda i: (0, 0)),
        ],
        out_specs=pl.BlockSpec((bn, d), lambda i: (i, 0)),
        out_shape=jax.ShapeDtypeStruct((n, d), jnp.float32),
    )(ph, pr, degm, h, W, b8, L)


def kernel(input_h, relation_embed, edges, W1, b1, L1, W2, b2, L2):
    n, d = input_h.shape
    e = edges.shape[0]

    per_chunkset = NW * CH
    n_chunks = NB * math.ceil(e / (per_chunkset * NB))
    ep = n_chunks * per_chunkset
    np_rows = NS * 128 * math.ceil((n + 1) / (NS * 128))

    pad = ep - e
    src = jnp.concatenate([edges[:, 0], jnp.zeros((pad,), jnp.int32)])
    rel = jnp.concatenate([edges[:, 1], jnp.zeros((pad,), jnp.int32)])
    dst = jnp.concatenate([edges[:, 2], jnp.full((pad,), n, jnp.int32)])
    src3 = src.reshape(NW, n_chunks, CH)
    rel3 = rel.reshape(NW, n_chunks, CH)
    dst3 = dst.reshape(NW, n_chunks, CH)

    pr, pdeg = _sc_segsum(n_chunks, np_rows, d, True, relation_embed, rel3, dst3)
    ph1 = _sc_segsum(n_chunks, np_rows, d, False, input_h, src3, dst3)

    deg = pdeg[0, :n] + pdeg[1, :n]
    degm = jnp.broadcast_to(jnp.maximum(deg, 1.0)[:, None], (n, d))
    b1p = jnp.broadcast_to(b1[None, :], (8, d))
    b2p = jnp.broadcast_to(b2[None, :], (8, d))

    out1 = _tc_layer(n, d, ph1, pr, degm, input_h, W1, b1p, L1)
    ph2 = _sc_segsum(n_chunks, np_rows, d, False, out1, src3, dst3)
    out2 = _tc_layer(n, d, ph2, pr, degm, out1, W2, b2p, L2)
    return out2


# speedup vs baseline: 1.1367x; 1.1367x over previous
import math

import jax
import jax.numpy as jnp
from jax import lax
from jax.experimental import pallas as pl
from jax.experimental.pallas import tpu as pltpu
from jax.experimental.pallas import tpu_sc as plsc

NC = 2
NS = 16
NW = NC * NS
CH = 128
NB = 2


def _sc_segsum(n_chunks, np_rows, d, with_deg, table, gidx, sidx):
    rows_per_tile = np_rows // NS
    zch = 64
    ngroups = n_chunks // NB

    out_type = [jax.ShapeDtypeStruct((NC, np_rows, d), jnp.float32)]
    if with_deg:
        out_type.append(jax.ShapeDtypeStruct((NC, np_rows), jnp.float32))

    scratch = [
        pltpu.VMEM((zch, d), jnp.float32),
        [pltpu.VMEM((CH,), jnp.int32) for _ in range(NB)],
        [pltpu.VMEM((CH,), jnp.int32) for _ in range(NB)],
        [pltpu.VMEM((CH, d), jnp.float32) for _ in range(NB)],
        pltpu.VMEM_SHARED((np_rows, d), jnp.float32),
        [pltpu.SemaphoreType.DMA for _ in range(NB)],
        [pltpu.SemaphoreType.DMA for _ in range(NB)],
        [pltpu.SemaphoreType.DMA for _ in range(NB)],
        [pltpu.SemaphoreType.DMA for _ in range(NB)],
    ]
    if with_deg:
        scratch += [
            pltpu.VMEM((CH,), jnp.float32),
            pltpu.VMEM((rows_per_tile,), jnp.float32),
            pltpu.VMEM_SHARED((np_rows,), jnp.float32),
            [pltpu.SemaphoreType.DMA for _ in range(NB)],
        ]

    def body(table_hbm, gidx_hbm, sidx_hbm, *refs):
        if with_deg:
            (out_hbm, outd_hbm, zrows, gib, sib, rows, acc,
             gsem, ssem, gisem, sisem, ones_v, zdeg_v, dacc, dsem) = refs
        else:
            (out_hbm, zrows, gib, sib, rows, acc,
             gsem, ssem, gisem, sisem) = refs

        cid = lax.axis_index("c")
        sid = lax.axis_index("s")
        wid = sid * NC + cid
        base = sid * rows_per_tile

        def zrow(i, _):
            for j in range(d // 16):
                zrows[i, pl.ds(j * 16, 16)] = jnp.zeros((16,), jnp.float32)
            return 0
        lax.fori_loop(0, zch, zrow, 0)
        for k in range(rows_per_tile // zch):
            pltpu.sync_copy(zrows, acc.at[pl.ds(base + k * zch, zch)])
        if with_deg:
            def zdeg(i, _):
                zdeg_v[pl.ds(i * 16, 16)] = jnp.zeros((16,), jnp.float32)
                return 0
            lax.fori_loop(0, rows_per_tile // 16, zdeg, 0)
            for j in range(CH // 16):
                ones_v[pl.ds(j * 16, 16)] = jnp.ones((16,), jnp.float32)
            pltpu.sync_copy(zdeg_v, dacc.at[pl.ds(base, rows_per_tile)])

        def load_gidx(j, b):
            return pltpu.async_copy(gidx_hbm.at[wid, j], gib[b], gisem[b])

        def load_sidx(j, b):
            return pltpu.async_copy(sidx_hbm.at[wid, j], sib[b], sisem[b])

        def start_gather(b):
            return pltpu.async_copy(table_hbm.at[gib[b]], rows[b], gsem[b])

        def wait_gather(b):
            pltpu.make_async_copy(table_hbm.at[gib[b]], rows[b], gsem[b]).wait()

        def start_scatter(b):
            return pltpu.async_copy(rows[b], acc.at[sib[b]], ssem[b], add=True)

        def wait_scatter(b):
            pltpu.make_async_copy(rows[b], acc.at[sib[b]], ssem[b]).wait()

        def start_deg(b):
            return pltpu.async_copy(ones_v, dacc.at[sib[b]], dsem[b], add=True)

        def wait_deg(b):
            pltpu.make_async_copy(ones_v, dacc.at[sib[b]], dsem[b]).wait()

        def wait_gidx(j, b):
            pltpu.make_async_copy(gidx_hbm.at[wid, j], gib[b], gisem[b]).wait()

        def wait_sidx(j, b):
            pltpu.make_async_copy(sidx_hbm.at[wid, j], sib[b], sisem[b]).wait()

        plsc.subcore_barrier()

        for b in range(NB):
            load_gidx(b, b)
            load_sidx(b, b)
        for b in range(NB):
            wait_gidx(b, b)
            start_gather(b)

        def step(jo, _):
            j0 = NB * jo
            for b in range(NB):
                wait_gather(b)
                load_gidx(j0 + b + NB, b)
                wait_sidx(j0 + b, b)
                start_scatter(b)
                if with_deg:
                    start_deg(b)
            for b in range(NB):
                wait_scatter(b)
                if with_deg:
                    wait_deg(b)
                load_sidx(j0 + b + NB, b)
                wait_gidx(j0 + b + NB, b)
                start_gather(b)
            return 0
        lax.fori_loop(0, ngroups - 1, step, 0)

        jl = n_chunks - NB
        for b in range(NB):
            wait_gather(b)
            wait_sidx(jl + b, b)
            start_scatter(b)
            if with_deg:
                start_deg(b)
        for b in range(NB):
            wait_scatter(b)
            if with_deg:
                wait_deg(b)

        plsc.subcore_barrier()

        for k in range(rows_per_tile // zch):
            pltpu.sync_copy(acc.at[pl.ds(base + k * zch, zch)],
                            out_hbm.at[cid, pl.ds(base + k * zch, zch)])
        if with_deg:
            pltpu.sync_copy(dacc.at[pl.ds(base, rows_per_tile)],
                            outd_hbm.at[cid, pl.ds(base, rows_per_tile)])

    mesh = plsc.VectorSubcoreMesh(core_axis_name="c", subcore_axis_name="s",
                                  num_cores=NC, num_subcores=NS)
    run = pl.kernel(body, out_type=out_type, mesh=mesh, scratch_types=scratch)
    res = run(table, gidx, sidx)
    return res if with_deg else res[0]


def _tc_layer(n, d, ph, pr, degm, h, W, b8, L):
    bn = 1000
    grid = n // bn

    def body(ph_ref, pr_ref, degm_ref, h_ref, W_ref, b_ref, L_ref, o_ref):
        agg = (ph_ref[0] + ph_ref[1] + pr_ref[0] + pr_ref[1]) / degm_ref[...]
        o = jnp.dot(agg, W_ref[...], preferred_element_type=jnp.float32)
        o = o + jnp.dot(h_ref[...], L_ref[...], preferred_element_type=jnp.float32)
        o = o + b_ref[0:1, :]
        o_ref[...] = jnp.maximum(o, 0.0)

    return pl.pallas_call(
        body,
        grid=(grid,),
        in_specs=[
            pl.BlockSpec((NC, bn, d), lambda i: (0, i, 0)),
            pl.BlockSpec((NC, bn, d), lambda i: (0, i, 0)),
            pl.BlockSpec((bn, d), lambda i: (i, 0)),
            pl.BlockSpec((bn, d), lambda i: (i, 0)),
            pl.BlockSpec((d, d), lambda i: (0, 0)),
            pl.BlockSpec((8, d), lambda i: (0, 0)),
            pl.BlockSpec((d, d), lambda i: (0, 0)),
        ],
        out_specs=pl.BlockSpec((bn, d), lambda i: (i, 0)),
        out_shape=jax.ShapeDtypeStruct((n, d), jnp.float32),
    )(ph, pr, degm, h, W, b8, L)


def kernel(input_h, relation_embed, edges, W1, b1, L1, W2, b2, L2):
    n, d = input_h.shape
    e = edges.shape[0]

    per_chunkset = NW * CH
    n_chunks = NB * math.ceil(e / (per_chunkset * NB))
    ep = n_chunks * per_chunkset
    np_rows = NS * 128 * math.ceil((n + 1) / (NS * 128))

    pad = ep - e
    src = jnp.concatenate([edges[:, 0], jnp.zeros((pad,), jnp.int32)])
    rel = jnp.concatenate([edges[:, 1], jnp.zeros((pad,), jnp.int32)])
    dst = jnp.concatenate([edges[:, 2], jnp.full((pad,), n, jnp.int32)])
    src3 = src.reshape(NW, n_chunks, CH)
    rel3 = rel.reshape(NW, n_chunks, CH)
    dst3 = dst.reshape(NW, n_chunks, CH)

    pr, pdeg = _sc_segsum(n_chunks, np_rows, d, True, relation_embed, rel3, dst3)
    ph1 = _sc_segsum(n_chunks, np_rows, d, False, input_h, src3, dst3)

    deg = pdeg[0, :n] + pdeg[1, :n]
    degm = jnp.broadcast_to(jnp.maximum(deg, 1.0)[:, None], (n, d))
    b1p = jnp.broadcast_to(b1[None, :], (8, d))
    b2p = jnp.broadcast_to(b2[None, :], (8, d))

    out1 = _tc_layer(n, d, ph1, pr, degm, input_h, W1, b1p, L1)
    ph2 = _sc_segsum(n_chunks, np_rows, d, False, out1, src3, dst3)
    out2 = _tc_layer(n, d, ph2, pr, degm, out1, W2, b2p, L2)
    return out2


# speedup vs baseline: 2.0731x; 1.8238x over previous
import math

import jax
import jax.numpy as jnp
from jax import lax
from jax.experimental import pallas as pl
from jax.experimental.pallas import tpu as pltpu
from jax.experimental.pallas import tpu_sc as plsc

NC = 2
NS = 16
NW = NC * NS
CH = 128
NB = 2


def _sc_segsum(nc0, nc1, np_rows, d, with_deg, table, gidx, sidx):
    rows_per_tile = np_rows // NS
    zch = 64

    out_type = [jax.ShapeDtypeStruct((NC, np_rows, d), jnp.float32)]
    if with_deg:
        out_type.append(jax.ShapeDtypeStruct((NC, np_rows), jnp.float32))

    scratch = [
        pltpu.VMEM((zch, d), jnp.float32),
        [pltpu.VMEM((CH,), jnp.int32) for _ in range(NB)],
        [pltpu.VMEM((CH,), jnp.int32) for _ in range(NB)],
        [pltpu.VMEM((CH, d), jnp.float32) for _ in range(NB)],
        pltpu.VMEM_SHARED((np_rows, d), jnp.float32),
        [pltpu.SemaphoreType.DMA for _ in range(NB)],
        [pltpu.SemaphoreType.DMA for _ in range(NB)],
        [pltpu.SemaphoreType.DMA for _ in range(NB)],
        [pltpu.SemaphoreType.DMA for _ in range(NB)],
    ]
    if with_deg:
        scratch += [
            pltpu.VMEM((CH,), jnp.float32),
            pltpu.VMEM((rows_per_tile,), jnp.float32),
            pltpu.VMEM_SHARED((np_rows,), jnp.float32),
            [pltpu.SemaphoreType.DMA for _ in range(NB)],
        ]

    def body(table_hbm, gidx_hbm, sidx_hbm, *refs):
        if with_deg:
            (out_hbm, outd_hbm, zrows, gib, sib, rows, acc,
             gsem, ssem, gisem, sisem, ones_v, zdeg_v, dacc, dsem) = refs
        else:
            (out_hbm, zrows, gib, sib, rows, acc,
             gsem, ssem, gisem, sisem) = refs

        cid = lax.axis_index("c")
        sid = lax.axis_index("s")
        base = sid * rows_per_tile
        base_ch = jnp.where(cid == 0, 0, nc0)
        n_me = jnp.where(cid == 0, nc0, nc1)
        ngroups = n_me // NB

        def zrow(i, _):
            for j in range(d // 16):
                zrows[i, pl.ds(j * 16, 16)] = jnp.zeros((16,), jnp.float32)
            return 0
        lax.fori_loop(0, zch, zrow, 0)
        for k in range(rows_per_tile // zch):
            pltpu.sync_copy(zrows, acc.at[pl.ds(base + k * zch, zch)])
        if with_deg:
            def zdeg(i, _):
                zdeg_v[pl.ds(i * 16, 16)] = jnp.zeros((16,), jnp.float32)
                return 0
            lax.fori_loop(0, rows_per_tile // 16, zdeg, 0)
            for j in range(CH // 16):
                ones_v[pl.ds(j * 16, 16)] = jnp.ones((16,), jnp.float32)
            pltpu.sync_copy(zdeg_v, dacc.at[pl.ds(base, rows_per_tile)])

        def load_gidx(j, b):
            return pltpu.async_copy(gidx_hbm.at[sid, base_ch + j], gib[b], gisem[b])

        def load_sidx(j, b):
            return pltpu.async_copy(sidx_hbm.at[sid, base_ch + j], sib[b], sisem[b])

        def start_gather(b):
            return pltpu.async_copy(table_hbm.at[gib[b]], rows[b], gsem[b])

        def wait_gather(b):
            pltpu.make_async_copy(table_hbm.at[gib[b]], rows[b], gsem[b]).wait()

        def start_scatter(b):
            return pltpu.async_copy(rows[b], acc.at[sib[b]], ssem[b], add=True)

        def wait_scatter(b):
            pltpu.make_async_copy(rows[b], acc.at[sib[b]], ssem[b]).wait()

        def start_deg(b):
            return pltpu.async_copy(ones_v, dacc.at[sib[b]], dsem[b], add=True)

        def wait_deg(b):
            pltpu.make_async_copy(ones_v, dacc.at[sib[b]], dsem[b]).wait()

        def wait_gidx(j, b):
            pltpu.make_async_copy(gidx_hbm.at[sid, base_ch + j], gib[b],
                                  gisem[b]).wait()

        def wait_sidx(j, b):
            pltpu.make_async_copy(sidx_hbm.at[sid, base_ch + j], sib[b],
                                  sisem[b]).wait()

        plsc.subcore_barrier()

        def main_work():
            for b in range(NB):
                load_gidx(b, b)
                load_sidx(b, b)
            for b in range(NB):
                wait_gidx(b, b)
                start_gather(b)

            def step(jo, _):
                j0 = NB * jo
                for b in range(NB):
                    wait_gather(b)
                    load_gidx(j0 + b + NB, b)
                    wait_sidx(j0 + b, b)
                    start_scatter(b)
                    if with_deg:
                        start_deg(b)
                for b in range(NB):
                    wait_scatter(b)
                    if with_deg:
                        wait_deg(b)
                    load_sidx(j0 + b + NB, b)
                    wait_gidx(j0 + b + NB, b)
                    start_gather(b)
                return 0
            lax.fori_loop(0, ngroups - 1, step, 0)

            jl = n_me - NB
            for b in range(NB):
                wait_gather(b)
                wait_sidx(jl + b, b)
                start_scatter(b)
                if with_deg:
                    start_deg(b)
            for b in range(NB):
                wait_scatter(b)
                if with_deg:
                    wait_deg(b)

        main_work()

        plsc.subcore_barrier()

        for k in range(rows_per_tile // zch):
            pltpu.sync_copy(acc.at[pl.ds(base + k * zch, zch)],
                            out_hbm.at[cid, pl.ds(base + k * zch, zch)])
        if with_deg:
            pltpu.sync_copy(dacc.at[pl.ds(base, rows_per_tile)],
                            outd_hbm.at[cid, pl.ds(base, rows_per_tile)])

    mesh = plsc.VectorSubcoreMesh(core_axis_name="c", subcore_axis_name="s",
                                  num_cores=NC, num_subcores=NS)
    run = pl.kernel(body, out_type=out_type, mesh=mesh, scratch_types=scratch)
    res = run(table, gidx, sidx)
    return res if with_deg else res[0]


def _tc_layer(n, d, ph, pr, degm, h, W, b8, L):
    bn = 1000
    grid = n // bn

    def body(ph_ref, pr_ref, degm_ref, h_ref, W_ref, b_ref, L_ref, o_ref):
        agg = (ph_ref[0] + ph_ref[1] + pr_ref[0] + pr_ref[1]) / degm_ref[...]
        o = jnp.dot(agg, W_ref[...], preferred_element_type=jnp.float32)
        o = o + jnp.dot(h_ref[...], L_ref[...], preferred_element_type=jnp.float32)
        o = o + b_ref[0:1, :]
        o_ref[...] = jnp.maximum(o, 0.0)

    return pl.pallas_call(
        body,
        grid=(grid,),
        in_specs=[
            pl.BlockSpec((NC, bn, d), lambda i: (0, i, 0)),
            pl.BlockSpec((NC, bn, d), lambda i: (0, i, 0)),
            pl.BlockSpec((bn, d), lambda i: (i, 0)),
            pl.BlockSpec((bn, d), lambda i: (i, 0)),
            pl.BlockSpec((d, d), lambda i: (0, 0)),
            pl.BlockSpec((8, d), lambda i: (0, 0)),
            pl.BlockSpec((d, d), lambda i: (0, 0)),
        ],
        out_specs=pl.BlockSpec((bn, d), lambda i: (i, 0)),
        out_shape=jax.ShapeDtypeStruct((n, d), jnp.float32),
    )(ph, pr, degm, h, W, b8, L)


def kernel(input_h, relation_embed, edges, W1, b1, L1, W2, b2, L2):
    n, d = input_h.shape
    e = edges.shape[0]

    nct_raw = math.ceil(e / (NS * CH))
    nc1 = max(NB, NB * round(nct_raw * 0.254 / NB))
    nc0 = max(NB, NB * math.ceil((nct_raw - nc1) / NB))
    nct = nc0 + nc1
    ep = NS * nct * CH
    np_rows = NS * 128 * math.ceil((n + 1) / (NS * 128))

    pad = ep - e
    src = jnp.concatenate([edges[:, 0], jnp.zeros((pad,), jnp.int32)])
    rel = jnp.concatenate([edges[:, 1], jnp.zeros((pad,), jnp.int32)])
    dst = jnp.concatenate([edges[:, 2], jnp.full((pad,), n, jnp.int32)])
    src3 = src.reshape(NS, nct, CH)
    rel3 = rel.reshape(NS, nct, CH)
    dst3 = dst.reshape(NS, nct, CH)

    pr, pdeg = _sc_segsum(nc0, nc1, np_rows, d, True, relation_embed, rel3, dst3)
    ph1 = _sc_segsum(nc0, nc1, np_rows, d, False, input_h, src3, dst3)

    deg = pdeg[0, :n] + pdeg[1, :n]
    degm = jnp.broadcast_to(jnp.maximum(deg, 1.0)[:, None], (n, d))
    b1p = jnp.broadcast_to(b1[None, :], (8, d))
    b2p = jnp.broadcast_to(b2[None, :], (8, d))

    out1 = _tc_layer(n, d, ph1, pr, degm, input_h, W1, b1p, L1)
    ph2 = _sc_segsum(nc0, nc1, np_rows, d, False, out1, src3, dst3)
    out2 = _tc_layer(n, d, ph2, pr, degm, out1, W2, b2p, L2)
    return out2
